# baseline (speedup 1.0000x reference)
import functools

import jax
import jax.numpy as jnp
from jax import lax
from jax.experimental import pallas as pl
from jax.experimental.pallas import tpu as pltpu
from jax.experimental.pallas import tpu_sc as plsc

DIM = 64
NREL = 1000
BATCH = 16384
NC = 2
NS = 16
NW = NC * NS
COLS_PER_W = BATCH // NW
L = 16
CH = 128
N_CHUNKS = COLS_PER_W // CH
CH_GROUPS = CH // L


def _sc_body(subT_hbm, objT_hbm, rela_hbm, diag_hbm, out_hbm,
             idx_v, dg_v0, dg_v1, sub_v0, obj_v0, sub_v1, obj_v1,
             out_v, sem0, sem1):
    wid = lax.axis_index("s") * NC + lax.axis_index("c")
    base = wid * COLS_PER_W

    bufs = ((dg_v0, sub_v0, obj_v0, sem0), (dg_v1, sub_v1, obj_v1, sem1))

    pltpu.sync_copy(rela_hbm.at[pl.ds(wid * N_CHUNKS, N_CHUNKS)], idx_v)

    def fire(k):
        dg_vb, sub_vb, obj_vb, semb = bufs[k % 2]
        cbase = base + k * CH
        return (
            pltpu.async_copy(diag_hbm.at[idx_v.at[k]], dg_vb, semb),
            pltpu.async_copy(subT_hbm.at[:, pl.ds(cbase, CH)], sub_vb, semb),
            pltpu.async_copy(objT_hbm.at[:, pl.ds(cbase, CH)], obj_vb, semb),
        )

    lane = jnp.arange(L, dtype=jnp.int32)

    def compute(k):
        dg_vb, sub_vb, obj_vb, _ = bufs[k % 2]

        def bgroup(bg, carry):
            row = bg * L + lane
            acc = None
            for d in range(DIM):
                s = sub_vb[d, pl.ds(bg * L, L)]
                o = obj_vb[d, pl.ds(bg * L, L)]
                r = plsc.load_gather(
                    dg_vb, [row, jnp.full((L,), d, jnp.int32)])
                p = s * o * r
                acc = p if acc is None else acc + p
            out_v[pl.ds(k * CH + bg * L, L)] = acc
            return carry

        lax.fori_loop(0, CH_GROUPS, bgroup, 0)

    pending = fire(0)
    for k in range(N_CHUNKS):
        nxt = fire(k + 1) if k + 1 < N_CHUNKS else None
        for cp in pending:
            cp.wait()
        compute(k)
        pending = nxt

    pltpu.sync_copy(out_v, out_hbm.at[pl.ds(base, COLS_PER_W)])


@functools.partial(
    pl.kernel,
    out_type=jax.ShapeDtypeStruct((BATCH,), jnp.float32),
    mesh=plsc.VectorSubcoreMesh(core_axis_name="c", subcore_axis_name="s"),
    compiler_params=pltpu.CompilerParams(needs_layout_passes=False,
                                         use_tc_tiling_on_sc=False),
    scratch_types=[
        pltpu.VMEM((N_CHUNKS, CH), jnp.int32),
        pltpu.VMEM((CH, DIM), jnp.float32),
        pltpu.VMEM((CH, DIM), jnp.float32),
        pltpu.VMEM((DIM, CH), jnp.float32),
        pltpu.VMEM((DIM, CH), jnp.float32),
        pltpu.VMEM((DIM, CH), jnp.float32),
        pltpu.VMEM((DIM, CH), jnp.float32),
        pltpu.VMEM((COLS_PER_W,), jnp.float32),
        pltpu.SemaphoreType.DMA,
        pltpu.SemaphoreType.DMA,
    ],
)
def _dist_mult_sc(subT_hbm, objT_hbm, rela_hbm, diag_hbm, out_hbm, *scratch):
    _sc_body(subT_hbm, objT_hbm, rela_hbm, diag_hbm, out_hbm, *scratch)


def kernel(sub_embed, obj_embed, rela, diag):
    return _dist_mult_sc(sub_embed.T, obj_embed.T,
                         rela.astype(jnp.int32).reshape(BATCH // CH, CH),
                         diag)
